# baseline (speedup 1.0000x reference)
import functools

import jax
import jax.numpy as jnp
from jax import lax
from jax.experimental import pallas as pl
from jax.experimental.pallas import tpu as pltpu
from jax.experimental.pallas import tpu_sc as plsc

N_NODES = 10000
N_EDGES = 320000
F = 128
NG = 100
NPG = 100

NC, NS = 2, 16
EPT = N_EDGES // NS
KC = 128
ROWS = 168
NCH = 160
GRP = 4
NSUP = NCH // (2 * GRP)
NACC = 10240
DUMP = N_NODES


@functools.cache
def _get_sc_agg():
    mesh = plsc.VectorSubcoreMesh(
        core_axis_name="c", subcore_axis_name="s",
        num_cores=NC, num_subcores=NS)
    return pl.kernel(
        _sc_agg_body,
        out_type=jax.ShapeDtypeStruct((2, N_NODES, F), jnp.float32),
        mesh=mesh,
        scratch_types=[
            pltpu.VMEM((GRP, KC), jnp.int32),
            pltpu.VMEM((GRP, KC), jnp.int32),
            pltpu.VMEM((GRP, KC), jnp.int32),
            pltpu.VMEM((GRP, KC), jnp.int32),
            pltpu.VMEM((KC, F), jnp.float32),
            pltpu.VMEM((KC, F), jnp.float32),
            pltpu.VMEM_SHARED((NACC, F), jnp.float32),
            pltpu.SemaphoreType.DMA,
            pltpu.SemaphoreType.DMA,
            pltpu.SemaphoreType.DMA,
            pltpu.SemaphoreType.DMA,
            pltpu.SemaphoreType.DMA,
            pltpu.SemaphoreType.DMA,
        ],
    )


def _sc_agg_body(x_hbm, ei_hbm, zeros_hbm, out_hbm,
                 wa_src, wa_dst, wb_src, wb_dst, rows_a, rows_b, agg_sh,
                 isem_a, isem_b, gsem_a, gsem_b, ssem_a, ssem_b):
    c = lax.axis_index("c")
    s = lax.axis_index("s")
    rpt = 624
    tail = N_NODES - NS * rpt

    pltpu.sync_copy(zeros_hbm.at[pl.ds(s * rpt, rpt)],
                    agg_sh.at[pl.ds(s * rpt, rpt)])
    @pl.when(s == 0)
    def _():
        pltpu.sync_copy(zeros_hbm.at[pl.ds(NS * rpt, tail)],
                        agg_sh.at[pl.ds(NS * rpt, tail)])

    sbase = (c * NS + s) * ROWS
    dbase = ((1 - c) * NS + s) * ROWS

    def win_refill(g, wsrc, wdst, sem):
        pltpu.async_copy(ei_hbm.at[pl.ds(sbase + g * GRP, GRP)], wsrc, sem)
        pltpu.async_copy(ei_hbm.at[pl.ds(dbase + g * GRP, GRP)], wdst, sem)

    def win_wait(wsrc, wdst, sem):
        pltpu.make_async_copy(ei_hbm.at[pl.ds(sbase, GRP)], wsrc, sem).wait()
        pltpu.make_async_copy(ei_hbm.at[pl.ds(dbase, GRP)], wdst, sem).wait()

    def gather_issue(wsrc, i, rows, sem):
        pltpu.async_copy(x_hbm.at[wsrc.at[i]], rows, sem)

    def gather_wait(rows, sem):
        pltpu.make_async_copy(x_hbm.at[wa_src.at[0]], rows, sem).wait()

    def scat_issue(wdst, i, rows, sem):
        pltpu.async_copy(rows, agg_sh.at[wdst.at[i]], sem, add=True)

    def scat_wait(rows, sem):
        pltpu.make_async_copy(rows, agg_sh.at[wa_dst.at[0]], sem).wait()

    pltpu.sync_copy(ei_hbm.at[pl.ds(sbase, GRP)], wa_src)
    pltpu.sync_copy(ei_hbm.at[pl.ds(dbase, GRP)], wa_dst)
    plsc.subcore_barrier()
    gather_issue(wa_src, 0, rows_a, gsem_a)

    def body(u, carry):
        gather_wait(rows_a, gsem_a)
        scat_issue(wa_dst, 0, rows_a, ssem_a)

        @pl.when(u > 0)
        def _():
            scat_wait(rows_b, ssem_b)
        win_refill(2 * u + 1, wb_src, wb_dst, isem_b)
        gather_issue(wa_src, 1, rows_b, gsem_b)
        gather_wait(rows_b, gsem_b)
        scat_issue(wa_dst, 1, rows_b, ssem_b)
        scat_wait(rows_a, ssem_a)
        gather_issue(wa_src, 2, rows_a, gsem_a)
        gather_wait(rows_a, gsem_a)
        scat_issue(wa_dst, 2, rows_a, ssem_a)
        scat_wait(rows_b, ssem_b)
        gather_issue(wa_src, 3, rows_b, gsem_b)
        gather_wait(rows_b, gsem_b)
        scat_issue(wa_dst, 3, rows_b, ssem_b)
        scat_wait(rows_a, ssem_a)
        win_wait(wb_src, wb_dst, isem_b)
        gather_issue(wb_src, 0, rows_a, gsem_a)
        gather_wait(rows_a, gsem_a)
        scat_issue(wb_dst, 0, rows_a, ssem_a)
        scat_wait(rows_b, ssem_b)
        win_refill(2 * u + 2, wa_src, wa_dst, isem_a)
        gather_issue(wb_src, 1, rows_b, gsem_b)
        gather_wait(rows_b, gsem_b)
        scat_issue(wb_dst, 1, rows_b, ssem_b)
        scat_wait(rows_a, ssem_a)
        gather_issue(wb_src, 2, rows_a, gsem_a)
        gather_wait(rows_a, gsem_a)
        scat_issue(wb_dst, 2, rows_a, ssem_a)
        scat_wait(rows_b, ssem_b)
        gather_issue(wb_src, 3, rows_b, gsem_b)
        gather_wait(rows_b, gsem_b)
        scat_issue(wb_dst, 3, rows_b, ssem_b)
        scat_wait(rows_a, ssem_a)
        win_wait(wa_src, wa_dst, isem_a)
        gather_issue(wa_src, 0, rows_a, gsem_a)
        return carry

    lax.fori_loop(0, NSUP, body, 0)
    gather_wait(rows_a, gsem_a)
    scat_wait(rows_b, ssem_b)
    plsc.subcore_barrier()
    pltpu.sync_copy(agg_sh.at[pl.ds(s * rpt, rpt)],
                    out_hbm.at[c, pl.ds(s * rpt, rpt)])
    @pl.when(s == 0)
    def _():
        pltpu.sync_copy(agg_sh.at[pl.ds(NS * rpt, tail)],
                        out_hbm.at[c, pl.ds(NS * rpt, tail)])


GB = 20
RB = GB * NPG


def _dense_body(x_r, af_r, ar_r, w1a_r, b1a_r, w2a_r, b2a_r,
                w1b_r, b1b_r, w2b_r, b2b_r, wl1_r, bl1_r, wl2_r, bl2_r,
                wa_r, ba_r, out_r):
    xb = x_r[...]

    def mlp(agg, w1, b1, w2, b2):
        h = xb + agg
        h = jnp.maximum(jnp.dot(h, w1, preferred_element_type=jnp.float32)
                        + b1, 0.0)
        return jnp.dot(h, w2, preferred_element_type=jnp.float32) + b2

    hf = mlp(af_r[...], w1a_r[...], b1a_r[...], w2a_r[...], b2a_r[...])
    hr = mlp(ar_r[...], w1b_r[...], b1b_r[...], w2b_r[...], b2b_r[...])

    gi = lax.broadcasted_iota(jnp.int32, (GB, RB), 0)
    ni = lax.broadcasted_iota(jnp.int32, (GB, RB), 1) // NPG
    sel = (gi == ni).astype(jnp.float32)
    ug = lax.broadcasted_iota(jnp.int32, (RB, GB), 0) // NPG
    gg = lax.broadcasted_iota(jnp.int32, (RB, GB), 1)
    unsel = (ug == gg).astype(jnp.float32)

    poolf = jnp.dot(sel, hf, preferred_element_type=jnp.float32) / 100.0
    poolr = jnp.dot(sel, hr, preferred_element_type=jnp.float32) / 100.0

    wl1 = wl1_r[...]
    gproj = (jnp.dot(poolf, wl1[128:256], preferred_element_type=jnp.float32)
             + jnp.dot(poolr, wl1[384:512],
                       preferred_element_type=jnp.float32))
    z = (jnp.dot(hf, wl1[0:128], preferred_element_type=jnp.float32)
         + jnp.dot(hr, wl1[256:384], preferred_element_type=jnp.float32)
         + jnp.dot(unsel, gproj, preferred_element_type=jnp.float32)
         + bl1_r[...])
    z = jnp.maximum(z, 0.0)

    wl2 = wl2_r[...]
    z2 = (jnp.dot(z, wl2[0:128], preferred_element_type=jnp.float32)
          + jnp.dot(xb, wl2[128:256], preferred_element_type=jnp.float32)
          + bl2_r[...])
    z2 = jnp.maximum(z2, 0.0)

    out_r[...] = jnp.dot(z2, wa_r[...],
                         preferred_element_type=jnp.float32) + ba_r[...]


def _full(shape):
    return pl.BlockSpec(shape, lambda i: (0, 0))


_dense = pl.pallas_call(
    _dense_body,
    grid=(N_NODES // RB,),
    in_specs=[
        pl.BlockSpec((RB, F), lambda i: (i, 0)),
        pl.BlockSpec((RB, F), lambda i: (i, 0)),
        pl.BlockSpec((RB, F), lambda i: (i, 0)),
        _full((F, F)), _full((1, F)), _full((F, F)), _full((1, F)),
        _full((F, F)), _full((1, F)), _full((F, F)), _full((1, F)),
        _full((4 * F, F)), _full((1, F)),
        _full((2 * F, F)), _full((1, F)),
        _full((F, 1)), _full((1, 1)),
    ],
    out_specs=pl.BlockSpec((RB, 1), lambda i: (i, 0)),
    out_shape=jax.ShapeDtypeStruct((N_NODES, 1), jnp.float32),
)


def _sample_body(lg_r, pen_r, gum_r, offs_r, samp_r, la_r):
    v = lg_r[...] - pen_r[...]
    m = jnp.max(v, axis=1, keepdims=True)
    e = jnp.exp(v - m)
    probs = e / jnp.sum(e, axis=1, keepdims=True)
    t = jnp.log(probs + 1e-20) + gum_r[...]
    tm = jnp.max(t, axis=1, keepdims=True)
    col = lax.broadcasted_iota(jnp.int32, (NG, NPG), 1)
    idx = jnp.min(jnp.where(t == tm, col, jnp.int32(2**30)), axis=1,
                  keepdims=True)
    p_sel = jnp.sum(jnp.where(col == idx, probs, 0.0), axis=1, keepdims=True)
    samp_r[...] = idx + offs_r[...]
    la_r[...] = jnp.log(p_sel)


_sample = pl.pallas_call(
    _sample_body,
    out_shape=(jax.ShapeDtypeStruct((NG, 1), jnp.int32),
               jax.ShapeDtypeStruct((NG, 1), jnp.float32)),
)


def kernel(x, edge_index, batch, mask, graph_id_offset,
           W1a, b1a, W2a, b2a, W1b, b1b, W2b, b2b,
           Wl1, bl1, Wl2, bl2, Wa, ba):
    zeros = jnp.zeros((N_NODES, F), jnp.float32)
    x_ext = jnp.concatenate(
        [x, jnp.zeros((NACC - N_NODES, F), jnp.float32)], axis=0)
    e3 = edge_index.astype(jnp.int32).reshape(2, NS, EPT)
    pad = jnp.full((2, NS, ROWS * KC - EPT), DUMP, jnp.int32)
    ei2 = jnp.concatenate([e3, pad], axis=2).reshape(2 * NS * ROWS, KC)
    agg = _get_sc_agg()(x_ext, ei2, zeros)

    logits = _dense(x, agg[0], agg[1],
                    W1a, b1a.reshape(1, F), W2a, b2a.reshape(1, F),
                    W1b, b1b.reshape(1, F), W2b, b2b.reshape(1, F),
                    Wl1, bl1.reshape(1, F), Wl2, bl2.reshape(1, F),
                    Wa, ba.reshape(1, 1))

    lg = logits.reshape(NG, NPG)
    pen = jnp.where(mask, 0.0, 1e10).astype(jnp.float32).reshape(NG, NPG)
    gum = jax.random.gumbel(jax.random.key(42), (NG, NPG), jnp.float32)
    samp, la = _sample(lg, pen, gum, graph_id_offset.reshape(NG, 1))
    return samp.reshape(NG), la.reshape(NG)


# speedup vs baseline: 1.1031x; 1.1031x over previous
import functools

import jax
import jax.numpy as jnp
from jax import lax
from jax.experimental import pallas as pl
from jax.experimental.pallas import tpu as pltpu
from jax.experimental.pallas import tpu_sc as plsc

N_NODES = 10000
N_EDGES = 320000
F = 128
NG = 100
NPG = 100

NC, NS = 2, 16
EPT = N_EDGES // NS
KC = 128
ROWS = 168
NCH = 160
GRP = 4
NSUP = NCH // (2 * GRP)
NACC = 10240
DUMP = N_NODES


@functools.cache
def _get_sc_agg():
    mesh = plsc.VectorSubcoreMesh(
        core_axis_name="c", subcore_axis_name="s",
        num_cores=NC, num_subcores=NS)
    return pl.kernel(
        _sc_agg_body,
        out_type=jax.ShapeDtypeStruct((2, N_NODES, F), jnp.float32),
        mesh=mesh,
        scratch_types=[
            pltpu.VMEM((KC,), jnp.int32),
            pltpu.VMEM((KC,), jnp.int32),
            pltpu.VMEM((KC,), jnp.int32),
            pltpu.VMEM((KC,), jnp.int32),
            pltpu.VMEM((KC, F), jnp.float32),
            pltpu.VMEM((KC, F), jnp.float32),
            pltpu.VMEM_SHARED((NACC, F), jnp.float32),
            pltpu.SemaphoreType.DMA,
            pltpu.SemaphoreType.DMA,
            pltpu.SemaphoreType.DMA,
            pltpu.SemaphoreType.DMA,
            pltpu.SemaphoreType.DMA,
            pltpu.SemaphoreType.DMA,
        ],
    )


def _sc_agg_body(x_hbm, ei_hbm, zeros_hbm, out_hbm,
                 src_a, dst_a, src_b, dst_b, rows_a, rows_b, agg_sh,
                 isem_a, isem_b, gsem_a, gsem_b, ssem_a, ssem_b):
    c = lax.axis_index("c")
    s = lax.axis_index("s")
    rpt = 624
    tail = N_NODES - NS * rpt

    pltpu.sync_copy(zeros_hbm.at[pl.ds(s * rpt, rpt)],
                    agg_sh.at[pl.ds(s * rpt, rpt)])
    @pl.when(s == 0)
    def _():
        pltpu.sync_copy(zeros_hbm.at[pl.ds(NS * rpt, tail)],
                        agg_sh.at[pl.ds(NS * rpt, tail)])

    sbase = (c * NS + s) * ROWS * KC
    dbase = ((1 - c) * NS + s) * ROWS * KC

    def idx_issue(j, srcv, dstv, sem):
        off = j * KC
        pltpu.async_copy(ei_hbm.at[pl.ds(sbase + off, KC)], srcv, sem)
        pltpu.async_copy(ei_hbm.at[pl.ds(dbase + off, KC)], dstv, sem)

    def idx_wait(srcv, dstv, sem):
        pltpu.make_async_copy(ei_hbm.at[pl.ds(sbase, KC)], srcv, sem).wait()
        pltpu.make_async_copy(ei_hbm.at[pl.ds(dbase, KC)], dstv, sem).wait()

    def gather_issue(srcv, rows, sem):
        pltpu.async_copy(x_hbm.at[srcv], rows, sem)

    def gather_wait(rows, sem):
        pltpu.make_async_copy(x_hbm.at[src_a], rows, sem).wait()

    def scat_issue(rows, dstv, sem):
        pltpu.async_copy(rows, agg_sh.at[dstv], sem, add=True)

    def scat_wait(rows, sem):
        pltpu.make_async_copy(rows, agg_sh.at[dst_a], sem).wait()

    idx_issue(0, src_a, dst_a, isem_a)
    plsc.subcore_barrier()
    idx_wait(src_a, dst_a, isem_a)
    gather_issue(src_a, rows_a, gsem_a)
    idx_issue(1, src_b, dst_b, isem_b)

    def body(g, carry):
        gather_wait(rows_a, gsem_a)
        scat_issue(rows_a, dst_a, ssem_a)

        @pl.when(g > 0)
        def _():
            scat_wait(rows_b, ssem_b)
            idx_issue(2 * g + 1, src_b, dst_b, isem_b)
        idx_wait(src_b, dst_b, isem_b)
        gather_issue(src_b, rows_b, gsem_b)
        gather_wait(rows_b, gsem_b)
        scat_issue(rows_b, dst_b, ssem_b)

        scat_wait(rows_a, ssem_a)

        @pl.when(g < NCH // 2 - 1)
        def _():
            idx_issue(2 * g + 2, src_a, dst_a, isem_a)
            idx_wait(src_a, dst_a, isem_a)
            gather_issue(src_a, rows_a, gsem_a)
        return carry

    lax.fori_loop(0, NCH // 2, body, 0)
    scat_wait(rows_b, ssem_b)
    plsc.subcore_barrier()
    pltpu.sync_copy(agg_sh.at[pl.ds(s * rpt, rpt)],
                    out_hbm.at[c, pl.ds(s * rpt, rpt)])
    @pl.when(s == 0)
    def _():
        pltpu.sync_copy(agg_sh.at[pl.ds(NS * rpt, tail)],
                        out_hbm.at[c, pl.ds(NS * rpt, tail)])


GB = 20
RB = GB * NPG


def _dense_body(x_r, af_r, ar_r, w1a_r, b1a_r, w2a_r, b2a_r,
                w1b_r, b1b_r, w2b_r, b2b_r, wl1_r, bl1_r, wl2_r, bl2_r,
                wa_r, ba_r, out_r):
    xb = x_r[...]

    def mlp(agg, w1, b1, w2, b2):
        h = xb + agg
        h = jnp.maximum(jnp.dot(h, w1, preferred_element_type=jnp.float32)
                        + b1, 0.0)
        return jnp.dot(h, w2, preferred_element_type=jnp.float32) + b2

    hf = mlp(af_r[...], w1a_r[...], b1a_r[...], w2a_r[...], b2a_r[...])
    hr = mlp(ar_r[...], w1b_r[...], b1b_r[...], w2b_r[...], b2b_r[...])

    gi = lax.broadcasted_iota(jnp.int32, (GB, RB), 0)
    ni = lax.broadcasted_iota(jnp.int32, (GB, RB), 1) // NPG
    sel = (gi == ni).astype(jnp.float32)
    ug = lax.broadcasted_iota(jnp.int32, (RB, GB), 0) // NPG
    gg = lax.broadcasted_iota(jnp.int32, (RB, GB), 1)
    unsel = (ug == gg).astype(jnp.float32)

    poolf = jnp.dot(sel, hf, preferred_element_type=jnp.float32) / 100.0
    poolr = jnp.dot(sel, hr, preferred_element_type=jnp.float32) / 100.0

    wl1 = wl1_r[...]
    gproj = (jnp.dot(poolf, wl1[128:256], preferred_element_type=jnp.float32)
             + jnp.dot(poolr, wl1[384:512],
                       preferred_element_type=jnp.float32))
    z = (jnp.dot(hf, wl1[0:128], preferred_element_type=jnp.float32)
         + jnp.dot(hr, wl1[256:384], preferred_element_type=jnp.float32)
         + jnp.dot(unsel, gproj, preferred_element_type=jnp.float32)
         + bl1_r[...])
    z = jnp.maximum(z, 0.0)

    wl2 = wl2_r[...]
    z2 = (jnp.dot(z, wl2[0:128], preferred_element_type=jnp.float32)
          + jnp.dot(xb, wl2[128:256], preferred_element_type=jnp.float32)
          + bl2_r[...])
    z2 = jnp.maximum(z2, 0.0)

    out_r[...] = jnp.dot(z2, wa_r[...],
                         preferred_element_type=jnp.float32) + ba_r[...]


def _full(shape):
    return pl.BlockSpec(shape, lambda i: (0, 0))


_dense = pl.pallas_call(
    _dense_body,
    grid=(N_NODES // RB,),
    in_specs=[
        pl.BlockSpec((RB, F), lambda i: (i, 0)),
        pl.BlockSpec((RB, F), lambda i: (i, 0)),
        pl.BlockSpec((RB, F), lambda i: (i, 0)),
        _full((F, F)), _full((1, F)), _full((F, F)), _full((1, F)),
        _full((F, F)), _full((1, F)), _full((F, F)), _full((1, F)),
        _full((4 * F, F)), _full((1, F)),
        _full((2 * F, F)), _full((1, F)),
        _full((F, 1)), _full((1, 1)),
    ],
    out_specs=pl.BlockSpec((RB, 1), lambda i: (i, 0)),
    out_shape=jax.ShapeDtypeStruct((N_NODES, 1), jnp.float32),
)


def _sample_body(lg_r, pen_r, gum_r, offs_r, samp_r, la_r):
    v = lg_r[...] - pen_r[...]
    m = jnp.max(v, axis=1, keepdims=True)
    e = jnp.exp(v - m)
    probs = e / jnp.sum(e, axis=1, keepdims=True)
    t = jnp.log(probs + 1e-20) + gum_r[...]
    tm = jnp.max(t, axis=1, keepdims=True)
    col = lax.broadcasted_iota(jnp.int32, (NG, NPG), 1)
    idx = jnp.min(jnp.where(t == tm, col, jnp.int32(2**30)), axis=1,
                  keepdims=True)
    p_sel = jnp.sum(jnp.where(col == idx, probs, 0.0), axis=1, keepdims=True)
    samp_r[...] = idx + offs_r[...]
    la_r[...] = jnp.log(p_sel)


_sample = pl.pallas_call(
    _sample_body,
    out_shape=(jax.ShapeDtypeStruct((NG, 1), jnp.int32),
               jax.ShapeDtypeStruct((NG, 1), jnp.float32)),
)


def kernel(x, edge_index, batch, mask, graph_id_offset,
           W1a, b1a, W2a, b2a, W1b, b1b, W2b, b2b,
           Wl1, bl1, Wl2, bl2, Wa, ba):
    zeros = jnp.zeros((N_NODES, F), jnp.float32)
    x_ext = jnp.concatenate(
        [x, jnp.zeros((NACC - N_NODES, F), jnp.float32)], axis=0)
    e3 = edge_index.astype(jnp.int32).reshape(2, NS, EPT)
    pad = jnp.full((2, NS, ROWS * KC - EPT), DUMP, jnp.int32)
    ei2 = jnp.concatenate([e3, pad], axis=2).reshape(2 * NS * ROWS * KC)
    agg = _get_sc_agg()(x_ext, ei2, zeros)

    logits = _dense(x, agg[0], agg[1],
                    W1a, b1a.reshape(1, F), W2a, b2a.reshape(1, F),
                    W1b, b1b.reshape(1, F), W2b, b2b.reshape(1, F),
                    Wl1, bl1.reshape(1, F), Wl2, bl2.reshape(1, F),
                    Wa, ba.reshape(1, 1))

    lg = logits.reshape(NG, NPG)
    pen = jnp.where(mask, 0.0, 1e10).astype(jnp.float32).reshape(NG, NPG)
    gum = jax.random.gumbel(jax.random.key(42), (NG, NPG), jnp.float32)
    samp, la = _sample(lg, pen, gum, graph_id_offset.reshape(NG, 1))
    return samp.reshape(NG), la.reshape(NG)


# speedup vs baseline: 1.6725x; 1.5162x over previous
import functools

import jax
import jax.numpy as jnp
from jax import lax
from jax.experimental import pallas as pl
from jax.experimental.pallas import tpu as pltpu
from jax.experimental.pallas import tpu_sc as plsc

N_NODES = 10000
N_EDGES = 320000
F = 128
NG = 100
NPG = 100

NC, NS = 2, 16
EPT = N_EDGES // NS
KC = 64
NCH = -(-EPT // KC) + (-(-EPT // KC)) % 2
EPAD = NCH * KC
NACC = 10240
DUMP = N_NODES


@functools.cache
def _get_sc_agg():
    mesh = plsc.VectorSubcoreMesh(
        core_axis_name="c", subcore_axis_name="s",
        num_cores=NC, num_subcores=NS)
    return pl.kernel(
        _sc_agg_body,
        out_type=jax.ShapeDtypeStruct((2, N_NODES, F), jnp.float32),
        mesh=mesh,
        scratch_types=[
            pltpu.VMEM((KC,), jnp.int32),
            pltpu.VMEM((KC,), jnp.int32),
            pltpu.VMEM((KC,), jnp.int32),
            pltpu.VMEM((KC,), jnp.int32),
            pltpu.VMEM((KC, F), jnp.float32),
            pltpu.VMEM((KC, F), jnp.float32),
            pltpu.VMEM_SHARED((NACC, F), jnp.float32),
            pltpu.SemaphoreType.DMA,
            pltpu.SemaphoreType.DMA,
            pltpu.SemaphoreType.DMA,
            pltpu.SemaphoreType.DMA,
            pltpu.SemaphoreType.DMA,
            pltpu.SemaphoreType.DMA,
        ],
    )


def _sc_agg_body(x_hbm, ei_hbm, zeros_hbm, out_hbm,
                 src_a, dst_a, src_b, dst_b, rows_a, rows_b, agg_sh,
                 isem_a, isem_b, gsem_a, gsem_b, ssem_a, ssem_b):
    c = lax.axis_index("c")
    s = lax.axis_index("s")
    rpt = 624
    tail = N_NODES - NS * rpt

    pltpu.sync_copy(zeros_hbm.at[pl.ds(s * rpt, rpt)],
                    agg_sh.at[pl.ds(s * rpt, rpt)])
    @pl.when(s == 0)
    def _():
        pltpu.sync_copy(zeros_hbm.at[pl.ds(NS * rpt, tail)],
                        agg_sh.at[pl.ds(NS * rpt, tail)])

    sbase = (c * NS + s) * EPAD
    dbase = ((1 - c) * NS + s) * EPAD

    def idx_issue(j, srcv, dstv, sem):
        off = j * KC
        pltpu.async_copy(ei_hbm.at[pl.ds(sbase + off, KC)], srcv, sem)
        pltpu.async_copy(ei_hbm.at[pl.ds(dbase + off, KC)], dstv, sem)

    def idx_wait(srcv, dstv, sem):
        pltpu.make_async_copy(ei_hbm.at[pl.ds(sbase, KC)], srcv, sem).wait()
        pltpu.make_async_copy(ei_hbm.at[pl.ds(dbase, KC)], dstv, sem).wait()

    def gather_issue(srcv, rows, sem):
        pltpu.async_copy(x_hbm.at[srcv], rows, sem)

    def gather_wait(rows, sem):
        pltpu.make_async_copy(x_hbm.at[src_a], rows, sem).wait()

    def scat_issue(rows, dstv, sem):
        pltpu.async_copy(rows, agg_sh.at[dstv], sem, add=True)

    def scat_wait(rows, sem):
        pltpu.make_async_copy(rows, agg_sh.at[dst_a], sem).wait()

    idx_issue(0, src_a, dst_a, isem_a)
    plsc.subcore_barrier()
    idx_wait(src_a, dst_a, isem_a)
    gather_issue(src_a, rows_a, gsem_a)
    idx_issue(1, src_b, dst_b, isem_b)

    def body(g, carry):
        gather_wait(rows_a, gsem_a)
        scat_issue(rows_a, dst_a, ssem_a)

        @pl.when(g > 0)
        def _():
            scat_wait(rows_b, ssem_b)
            idx_issue(2 * g + 1, src_b, dst_b, isem_b)
        idx_wait(src_b, dst_b, isem_b)
        gather_issue(src_b, rows_b, gsem_b)
        gather_wait(rows_b, gsem_b)
        scat_issue(rows_b, dst_b, ssem_b)

        scat_wait(rows_a, ssem_a)

        @pl.when(g < NCH // 2 - 1)
        def _():
            idx_issue(2 * g + 2, src_a, dst_a, isem_a)
            idx_wait(src_a, dst_a, isem_a)
            gather_issue(src_a, rows_a, gsem_a)
        return carry

    lax.fori_loop(0, NCH // 2, body, 0)
    scat_wait(rows_b, ssem_b)
    plsc.subcore_barrier()
    pltpu.sync_copy(agg_sh.at[pl.ds(s * rpt, rpt)],
                    out_hbm.at[c, pl.ds(s * rpt, rpt)])
    @pl.when(s == 0)
    def _():
        pltpu.sync_copy(agg_sh.at[pl.ds(NS * rpt, tail)],
                        out_hbm.at[c, pl.ds(NS * rpt, tail)])


GB = 20
RB = GB * NPG


def _dense_body(x_r, af_r, ar_r, w1a_r, b1a_r, w2a_r, b2a_r,
                w1b_r, b1b_r, w2b_r, b2b_r, wl1_r, bl1_r, wl2_r, bl2_r,
                wa_r, ba_r, out_r):
    xb = x_r[...]

    def mlp(agg, w1, b1, w2, b2):
        h = xb + agg
        h = jnp.maximum(jnp.dot(h, w1, preferred_element_type=jnp.float32)
                        + b1, 0.0)
        return jnp.dot(h, w2, preferred_element_type=jnp.float32) + b2

    hf = mlp(af_r[...], w1a_r[...], b1a_r[...], w2a_r[...], b2a_r[...])
    hr = mlp(ar_r[...], w1b_r[...], b1b_r[...], w2b_r[...], b2b_r[...])

    gi = lax.broadcasted_iota(jnp.int32, (GB, RB), 0)
    ni = lax.broadcasted_iota(jnp.int32, (GB, RB), 1) // NPG
    sel = (gi == ni).astype(jnp.float32)
    ug = lax.broadcasted_iota(jnp.int32, (RB, GB), 0) // NPG
    gg = lax.broadcasted_iota(jnp.int32, (RB, GB), 1)
    unsel = (ug == gg).astype(jnp.float32)

    poolf = jnp.dot(sel, hf, preferred_element_type=jnp.float32) / 100.0
    poolr = jnp.dot(sel, hr, preferred_element_type=jnp.float32) / 100.0

    wl1 = wl1_r[...]
    gproj = (jnp.dot(poolf, wl1[128:256], preferred_element_type=jnp.float32)
             + jnp.dot(poolr, wl1[384:512],
                       preferred_element_type=jnp.float32))
    z = (jnp.dot(hf, wl1[0:128], preferred_element_type=jnp.float32)
         + jnp.dot(hr, wl1[256:384], preferred_element_type=jnp.float32)
         + jnp.dot(unsel, gproj, preferred_element_type=jnp.float32)
         + bl1_r[...])
    z = jnp.maximum(z, 0.0)

    wl2 = wl2_r[...]
    z2 = (jnp.dot(z, wl2[0:128], preferred_element_type=jnp.float32)
          + jnp.dot(xb, wl2[128:256], preferred_element_type=jnp.float32)
          + bl2_r[...])
    z2 = jnp.maximum(z2, 0.0)

    out_r[...] = jnp.dot(z2, wa_r[...],
                         preferred_element_type=jnp.float32) + ba_r[...]


def _full(shape):
    return pl.BlockSpec(shape, lambda i: (0, 0))


_dense = pl.pallas_call(
    _dense_body,
    grid=(N_NODES // RB,),
    in_specs=[
        pl.BlockSpec((RB, F), lambda i: (i, 0)),
        pl.BlockSpec((RB, F), lambda i: (i, 0)),
        pl.BlockSpec((RB, F), lambda i: (i, 0)),
        _full((F, F)), _full((1, F)), _full((F, F)), _full((1, F)),
        _full((F, F)), _full((1, F)), _full((F, F)), _full((1, F)),
        _full((4 * F, F)), _full((1, F)),
        _full((2 * F, F)), _full((1, F)),
        _full((F, 1)), _full((1, 1)),
    ],
    out_specs=pl.BlockSpec((RB, 1), lambda i: (i, 0)),
    out_shape=jax.ShapeDtypeStruct((N_NODES, 1), jnp.float32),
)


def _sample_body(lg_r, pen_r, gum_r, offs_r, samp_r, la_r):
    v = lg_r[...] - pen_r[...]
    m = jnp.max(v, axis=1, keepdims=True)
    e = jnp.exp(v - m)
    probs = e / jnp.sum(e, axis=1, keepdims=True)
    t = jnp.log(probs + 1e-20) + gum_r[...]
    tm = jnp.max(t, axis=1, keepdims=True)
    col = lax.broadcasted_iota(jnp.int32, (NG, NPG), 1)
    idx = jnp.min(jnp.where(t == tm, col, jnp.int32(2**30)), axis=1,
                  keepdims=True)
    p_sel = jnp.sum(jnp.where(col == idx, probs, 0.0), axis=1, keepdims=True)
    samp_r[...] = idx + offs_r[...]
    la_r[...] = jnp.log(p_sel)


_sample = pl.pallas_call(
    _sample_body,
    out_shape=(jax.ShapeDtypeStruct((NG, 1), jnp.int32),
               jax.ShapeDtypeStruct((NG, 1), jnp.float32)),
)


def kernel(x, edge_index, batch, mask, graph_id_offset,
           W1a, b1a, W2a, b2a, W1b, b1b, W2b, b2b,
           Wl1, bl1, Wl2, bl2, Wa, ba):
    zeros = jnp.zeros((N_NODES, F), jnp.float32)
    x_ext = jnp.concatenate(
        [x, jnp.zeros((NACC - N_NODES, F), jnp.float32)], axis=0)
    e3 = edge_index.astype(jnp.int32).reshape(2, NS, EPT)
    pad = jnp.full((2, NS, EPAD - EPT), DUMP, jnp.int32)
    ei2 = jnp.concatenate([e3, pad], axis=2).reshape(2 * NS * EPAD)
    agg = _get_sc_agg()(x_ext, ei2, zeros)

    logits = _dense(x, agg[0], agg[1],
                    W1a, b1a.reshape(1, F), W2a, b2a.reshape(1, F),
                    W1b, b1b.reshape(1, F), W2b, b2b.reshape(1, F),
                    Wl1, bl1.reshape(1, F), Wl2, bl2.reshape(1, F),
                    Wa, ba.reshape(1, 1))

    lg = logits.reshape(NG, NPG)
    pen = jnp.where(mask, 0.0, 1e10).astype(jnp.float32).reshape(NG, NPG)
    gum = jax.random.gumbel(jax.random.key(42), (NG, NPG), jnp.float32)
    samp, la = _sample(lg, pen, gum, graph_id_offset.reshape(NG, 1))
    return samp.reshape(NG), la.reshape(NG)


# speedup vs baseline: 3.9793x; 2.3792x over previous
import functools

import jax
import jax.numpy as jnp
from jax import lax
from jax.experimental import pallas as pl
from jax.experimental.pallas import tpu as pltpu
from jax.experimental.pallas import tpu_sc as plsc

N_NODES = 10000
N_EDGES = 320000
F = 128
NG = 100
NPG = 100

NC, NS = 2, 16
EPT = N_EDGES // NS
KC = 80
NCH = EPT // KC


@functools.cache
def _get_sc_agg():
    mesh = plsc.VectorSubcoreMesh(
        core_axis_name="c", subcore_axis_name="s",
        num_cores=NC, num_subcores=NS)
    return pl.kernel(
        _sc_agg_body,
        out_type=jax.ShapeDtypeStruct((2, N_NODES, F), jnp.float32),
        mesh=mesh,
        scratch_types=(
            [pltpu.VMEM((KC,), jnp.int32)] * 8
            + [pltpu.VMEM((KC, F), jnp.float32)] * 4
            + [pltpu.VMEM_SHARED((N_NODES, F), jnp.float32)]
            + [pltpu.SemaphoreType.DMA] * 12
        ),
    )


def _sc_agg_body(x_hbm, ei_hbm, zeros_hbm, out_hbm,
                 src_a, dst_a, src_b, dst_b, src_c, dst_c, src_d, dst_d,
                 rows_a, rows_b, rows_c, rows_d, agg_sh,
                 isem_a, isem_b, isem_c, isem_d,
                 gsem_a, gsem_b, gsem_c, gsem_d,
                 ssem_a, ssem_b, ssem_c, ssem_d):
    c = lax.axis_index("c")
    s = lax.axis_index("s")
    rpt = 624
    tail = N_NODES - NS * rpt

    pltpu.sync_copy(zeros_hbm.at[pl.ds(s * rpt, rpt)],
                    agg_sh.at[pl.ds(s * rpt, rpt)])
    @pl.when(s == 0)
    def _():
        pltpu.sync_copy(zeros_hbm.at[pl.ds(NS * rpt, tail)],
                        agg_sh.at[pl.ds(NS * rpt, tail)])

    sbase = (c * NS + s) * EPT
    dbase = ((1 - c) * NS + s) * EPT

    def idx_issue(j, st):
        off = j * KC
        pltpu.async_copy(ei_hbm.at[pl.ds(sbase + off, KC)], st[0], st[3])
        pltpu.async_copy(ei_hbm.at[pl.ds(dbase + off, KC)], st[1], st[3])

    def idx_wait(st):
        pltpu.make_async_copy(ei_hbm.at[pl.ds(sbase, KC)], st[0], st[3]).wait()
        pltpu.make_async_copy(ei_hbm.at[pl.ds(dbase, KC)], st[1], st[3]).wait()

    def gather_issue(st):
        pltpu.async_copy(x_hbm.at[st[0]], st[2], st[4])

    def gather_wait(st):
        pltpu.make_async_copy(x_hbm.at[src_a], st[2], st[4]).wait()

    def scat_issue(st):
        pltpu.async_copy(st[2], agg_sh.at[st[1]], st[5], add=True)

    def scat_wait(st):
        pltpu.make_async_copy(st[2], agg_sh.at[dst_a], st[5]).wait()

    sets = [(src_a, dst_a, rows_a, isem_a, gsem_a, ssem_a),
            (src_b, dst_b, rows_b, isem_b, gsem_b, ssem_b),
            (src_c, dst_c, rows_c, isem_c, gsem_c, ssem_c),
            (src_d, dst_d, rows_d, isem_d, gsem_d, ssem_d)]

    plsc.subcore_barrier()
    idx_issue(0, sets[0])
    idx_issue(1, sets[1])
    idx_wait(sets[0])
    gather_issue(sets[0])
    idx_wait(sets[1])
    gather_issue(sets[1])

    nsup = NCH // 4

    def body(u, carry):
        t0 = 4 * u
        for pos in range(4):
            cur = sets[pos]
            p2 = sets[(pos + 2) % 4]
            if pos < 2:
                @pl.when(u > 0)
                def _(p2=p2):
                    scat_wait(p2)
                idx_issue(t0 + pos + 2, p2)
                gather_wait(cur)
                scat_issue(cur)
                idx_wait(p2)
                gather_issue(p2)
            else:
                scat_wait(p2)
                gather_wait(cur)
                scat_issue(cur)
                idx_issue(t0 + pos + 2, p2)
                idx_wait(p2)
                gather_issue(p2)
        return carry

    lax.fori_loop(0, nsup, body, 0)
    scat_wait(sets[2])
    gather_wait(sets[0])
    scat_issue(sets[0])
    scat_wait(sets[3])
    gather_wait(sets[1])
    scat_issue(sets[1])
    scat_wait(sets[0])
    scat_wait(sets[1])
    plsc.subcore_barrier()
    pltpu.sync_copy(agg_sh.at[pl.ds(s * rpt, rpt)],
                    out_hbm.at[c, pl.ds(s * rpt, rpt)])
    @pl.when(s == 0)
    def _():
        pltpu.sync_copy(agg_sh.at[pl.ds(NS * rpt, tail)],
                        out_hbm.at[c, pl.ds(NS * rpt, tail)])


GB = 20
RB = GB * NPG


def _dense_body(x_r, af_r, ar_r, w1a_r, b1a_r, w2a_r, b2a_r,
                w1b_r, b1b_r, w2b_r, b2b_r, wl1_r, bl1_r, wl2_r, bl2_r,
                wa_r, ba_r, out_r):
    xb = x_r[...]

    def mlp(agg, w1, b1, w2, b2):
        h = xb + agg
        h = jnp.maximum(jnp.dot(h, w1, preferred_element_type=jnp.float32)
                        + b1, 0.0)
        return jnp.dot(h, w2, preferred_element_type=jnp.float32) + b2

    hf = mlp(af_r[...], w1a_r[...], b1a_r[...], w2a_r[...], b2a_r[...])
    hr = mlp(ar_r[...], w1b_r[...], b1b_r[...], w2b_r[...], b2b_r[...])

    gi = lax.broadcasted_iota(jnp.int32, (GB, RB), 0)
    ni = lax.broadcasted_iota(jnp.int32, (GB, RB), 1) // NPG
    sel = (gi == ni).astype(jnp.float32)
    ug = lax.broadcasted_iota(jnp.int32, (RB, GB), 0) // NPG
    gg = lax.broadcasted_iota(jnp.int32, (RB, GB), 1)
    unsel = (ug == gg).astype(jnp.float32)

    poolf = jnp.dot(sel, hf, preferred_element_type=jnp.float32) / 100.0
    poolr = jnp.dot(sel, hr, preferred_element_type=jnp.float32) / 100.0

    wl1 = wl1_r[...]
    gproj = (jnp.dot(poolf, wl1[128:256], preferred_element_type=jnp.float32)
             + jnp.dot(poolr, wl1[384:512],
                       preferred_element_type=jnp.float32))
    z = (jnp.dot(hf, wl1[0:128], preferred_element_type=jnp.float32)
         + jnp.dot(hr, wl1[256:384], preferred_element_type=jnp.float32)
         + jnp.dot(unsel, gproj, preferred_element_type=jnp.float32)
         + bl1_r[...])
    z = jnp.maximum(z, 0.0)

    wl2 = wl2_r[...]
    z2 = (jnp.dot(z, wl2[0:128], preferred_element_type=jnp.float32)
          + jnp.dot(xb, wl2[128:256], preferred_element_type=jnp.float32)
          + bl2_r[...])
    z2 = jnp.maximum(z2, 0.0)

    out_r[...] = jnp.dot(z2, wa_r[...],
                         preferred_element_type=jnp.float32) + ba_r[...]


def _full(shape):
    return pl.BlockSpec(shape, lambda i: (0, 0))


_dense = pl.pallas_call(
    _dense_body,
    grid=(N_NODES // RB,),
    in_specs=[
        pl.BlockSpec((RB, F), lambda i: (i, 0)),
        pl.BlockSpec((RB, F), lambda i: (i, 0)),
        pl.BlockSpec((RB, F), lambda i: (i, 0)),
        _full((F, F)), _full((1, F)), _full((F, F)), _full((1, F)),
        _full((F, F)), _full((1, F)), _full((F, F)), _full((1, F)),
        _full((4 * F, F)), _full((1, F)),
        _full((2 * F, F)), _full((1, F)),
        _full((F, 1)), _full((1, 1)),
    ],
    out_specs=pl.BlockSpec((RB, 1), lambda i: (i, 0)),
    out_shape=jax.ShapeDtypeStruct((N_NODES, 1), jnp.float32),
)


def _sample_body(lg_r, pen_r, gum_r, offs_r, samp_r, la_r):
    v = lg_r[...] - pen_r[...]
    m = jnp.max(v, axis=1, keepdims=True)
    e = jnp.exp(v - m)
    probs = e / jnp.sum(e, axis=1, keepdims=True)
    t = jnp.log(probs + 1e-20) + gum_r[...]
    tm = jnp.max(t, axis=1, keepdims=True)
    col = lax.broadcasted_iota(jnp.int32, (NG, NPG), 1)
    idx = jnp.min(jnp.where(t == tm, col, jnp.int32(2**30)), axis=1,
                  keepdims=True)
    p_sel = jnp.sum(jnp.where(col == idx, probs, 0.0), axis=1, keepdims=True)
    samp_r[...] = idx + offs_r[...]
    la_r[...] = jnp.log(p_sel)


_sample = pl.pallas_call(
    _sample_body,
    out_shape=(jax.ShapeDtypeStruct((NG, 1), jnp.int32),
               jax.ShapeDtypeStruct((NG, 1), jnp.float32)),
)


def kernel(x, edge_index, batch, mask, graph_id_offset,
           W1a, b1a, W2a, b2a, W1b, b1b, W2b, b2b,
           Wl1, bl1, Wl2, bl2, Wa, ba):
    zeros = jnp.zeros((N_NODES, F), jnp.float32)
    ei2 = edge_index.astype(jnp.int32).reshape(2 * N_EDGES)
    agg = _get_sc_agg()(x, ei2, zeros)

    logits = _dense(x, agg[0], agg[1],
                    W1a, b1a.reshape(1, F), W2a, b2a.reshape(1, F),
                    W1b, b1b.reshape(1, F), W2b, b2b.reshape(1, F),
                    Wl1, bl1.reshape(1, F), Wl2, bl2.reshape(1, F),
                    Wa, ba.reshape(1, 1))

    lg = logits.reshape(NG, NPG)
    pen = jnp.where(mask, 0.0, 1e10).astype(jnp.float32).reshape(NG, NPG)
    gum = jax.random.gumbel(jax.random.key(42), (NG, NPG), jnp.float32)
    samp, la = _sample(lg, pen, gum, graph_id_offset.reshape(NG, 1))
    return samp.reshape(NG), la.reshape(NG)
